# baseline (speedup 1.0000x reference)
import functools

import jax
import jax.numpy as jnp
from jax import lax
from jax.experimental import pallas as pl
from jax.experimental.pallas import tpu as pltpu
from jax.experimental.pallas import tpu_sc as plsc

VOCAB = 1000000
MAX_LEN = 200
EMBED_DIM = 64
BATCH = 4096

NUM_CORES = 2
NUM_SUBCORES = 16
NW = NUM_CORES * NUM_SUBCORES
CB = 4
SBB = CB * 128
NSB = MAX_LEN * (BATCH // SBB)
SB_PER_W = NSB // NW
GIDX = 128
NG = SBB // GIDX
DG = EMBED_DIM // 16
C4 = BATCH // SBB


def _build():
  mesh = plsc.VectorSubcoreMesh(core_axis_name="c", subcore_axis_name="s")

  @functools.partial(
      pl.kernel,
      mesh=mesh,
      compiler_params=pltpu.CompilerParams(use_tc_tiling_on_sc=False,
                                           needs_layout_passes=False),
      out_type=jax.ShapeDtypeStruct((MAX_LEN, 8, BATCH // 128, 1024),
                                    jnp.float32),
      scratch_types=[
          pltpu.VMEM((NG, GIDX), jnp.int32),
          pltpu.VMEM((NG, GIDX), jnp.int32),
          pltpu.VMEM((SBB, EMBED_DIM), jnp.float32),
          pltpu.VMEM((SBB, EMBED_DIM), jnp.float32),
          pltpu.VMEM((8, CB, 1024), jnp.float32),
          pltpu.VMEM((MAX_LEN, EMBED_DIM), jnp.float32),
          pltpu.SemaphoreType.DMA,
          pltpu.SemaphoreType.DMA,
          pltpu.SemaphoreType.DMA,
      ],
  )
  def emb_kernel(idx_hbm, table_hbm, pos_hbm, out_hbm,
                 idx_a, idx_b, g_a, g_b, tbuf, pos_v,
                 gsem_a, gsem_b, wsem):
    wid = lax.axis_index("s") * NUM_CORES + lax.axis_index("c")
    sb0 = wid * SB_PER_W

    pltpu.sync_copy(pos_hbm, pos_v)

    iota16 = lax.iota(jnp.int32, 16)
    rowvecs = [(dg * 16 + iota16) >> 3 for dg in range(DG)]
    colvecs = [((dg * 16 + iota16) & 7) * 128 for dg in range(DG)]

    bufs = ((idx_a, g_a, gsem_a), (idx_b, g_b, gsem_b))

    def fire(buf, i):
      idx_v, gbuf, gsem = buf
      pltpu.sync_copy(idx_hbm.at[pl.ds((sb0 + i) * NG, NG)], idx_v)
      for js in range(NG):
        pltpu.async_copy(
            table_hbm.at[idx_v.at[js]],
            gbuf.at[pl.ds(js * GIDX, GIDX)], gsem)

    def wait_write():
      pltpu.make_async_copy(tbuf, out_hbm.at[0, :, pl.ds(0, CB), :],
                            wsem).wait()

    def process(buf, i, first):
      idx_v, gbuf, gsem = buf
      sbid = sb0 + i
      l = sbid // C4
      c_start = (sbid - l * C4) * CB
      for js in range(NG):
        pltpu.make_async_copy(
            table_hbm.at[idx_v.at[js]],
            gbuf.at[pl.ds(js * GIDX, GIDX)], gsem).wait()
      if not first:
        wait_write()
      posvecs = [pos_v[l, pl.ds(dg * 16, 16)] for dg in range(DG)]

      @plsc.parallel_loop(0, SBB, unroll=8)
      def jj_body(jj):
        cl = jnp.full((16,), jj >> 7, jnp.int32)
        lane = jj & 127
        for dg in range(DG):
          val = gbuf[jj, pl.ds(dg * 16, 16)] + posvecs[dg]
          plsc.store_scatter(tbuf, [rowvecs[dg], cl, colvecs[dg] + lane], val)

      pltpu.async_copy(tbuf, out_hbm.at[l, :, pl.ds(c_start, CB), :], wsem)

    fire(bufs[0], 0)
    fire(bufs[1], 1)

    process(bufs[0], 0, True)
    fire(bufs[0], 2)
    process(bufs[1], 1, False)
    fire(bufs[1], 3)

    def pair_body(i, carry):
      k = 2 * i
      process(bufs[0], k, False)
      fire(bufs[0], k + 2)
      process(bufs[1], k + 1, False)
      fire(bufs[1], k + 3)
      return carry

    lax.fori_loop(1, SB_PER_W // 2 - 1, pair_body, 0)

    last = SB_PER_W - 2
    process(bufs[0], last, False)
    process(bufs[1], last + 1, False)
    wait_write()

  return emb_kernel


_emb = _build()


def kernel(inputs, token_table, pos_table):
  idx2d = inputs.astype(jnp.int32).T.reshape(-1, GIDX)
  W = _emb(idx2d, token_table, pos_table)
  W5 = W.reshape(MAX_LEN, 8, BATCH // 128, 8, 128)
  return W5.transpose(2, 4, 0, 1, 3).reshape(BATCH, MAX_LEN, EMBED_DIM)


# speedup vs baseline: 1.6670x; 1.6670x over previous
import functools

import jax
import jax.numpy as jnp
from jax import lax
from jax.experimental import pallas as pl
from jax.experimental.pallas import tpu as pltpu
from jax.experimental.pallas import tpu_sc as plsc

VOCAB = 1000000
MAX_LEN = 200
EMBED_DIM = 64
BATCH = 4096

NUM_CORES = 2
NUM_SUBCORES = 16
NW = NUM_CORES * NUM_SUBCORES
CB = 4
SBB = CB * 128
NSB = MAX_LEN * (BATCH // SBB)
SB_PER_W = NSB // NW
GIDX = 128
NG = SBB // GIDX
DG = EMBED_DIM // 16
C4 = BATCH // SBB


def _build():
  mesh = plsc.VectorSubcoreMesh(core_axis_name="c", subcore_axis_name="s")

  @functools.partial(
      pl.kernel,
      mesh=mesh,
      compiler_params=pltpu.CompilerParams(use_tc_tiling_on_sc=False,
                                           needs_layout_passes=False),
      out_type=jax.ShapeDtypeStruct((MAX_LEN, 8, BATCH // 128, 8, 128),
                                    jnp.float32),
      scratch_types=[
          pltpu.VMEM((NG, GIDX), jnp.int32),
          pltpu.VMEM((NG, GIDX), jnp.int32),
          pltpu.VMEM((SBB, EMBED_DIM), jnp.float32),
          pltpu.VMEM((SBB, EMBED_DIM), jnp.float32),
          pltpu.VMEM((8, CB, 8, 144), jnp.float32),
          pltpu.VMEM((MAX_LEN, EMBED_DIM), jnp.float32),
          pltpu.SemaphoreType.DMA,
          pltpu.SemaphoreType.DMA,
          pltpu.SemaphoreType.DMA,
      ],
  )
  def emb_kernel(idx_hbm, table_hbm, pos_hbm, out_hbm,
                 idx_a, idx_b, g_a, g_b, tbuf, pos_v,
                 gsem_a, gsem_b, wsem):
    wid = lax.axis_index("s") * NUM_CORES + lax.axis_index("c")
    sb0 = wid * SB_PER_W

    pltpu.sync_copy(pos_hbm, pos_v)

    iota16 = lax.iota(jnp.int32, 16)
    gvecs = [(dg * 16 + iota16) >> 3 for dg in range(DG)]
    rvecs = [(dg * 16 + iota16) & 7 for dg in range(DG)]

    bufs = ((idx_a, g_a, gsem_a), (idx_b, g_b, gsem_b))

    def fire(buf, i):
      idx_v, gbuf, gsem = buf
      pltpu.sync_copy(idx_hbm.at[pl.ds((sb0 + i) * NG, NG)], idx_v)
      for js in range(NG):
        pltpu.async_copy(
            table_hbm.at[idx_v.at[js]],
            gbuf.at[pl.ds(js * GIDX, GIDX)], gsem)

    def wait_write():
      pltpu.make_async_copy(tbuf.at[:, :, :, pl.ds(0, 128)],
                            out_hbm.at[0, :, pl.ds(0, CB), :, :],
                            wsem).wait()

    def process(buf, i, first):
      idx_v, gbuf, gsem = buf
      sbid = sb0 + i
      l = sbid // C4
      c_start = (sbid - l * C4) * CB
      for js in range(NG):
        pltpu.make_async_copy(
            table_hbm.at[idx_v.at[js]],
            gbuf.at[pl.ds(js * GIDX, GIDX)], gsem).wait()
      if not first:
        wait_write()
      posvecs = [pos_v[l, pl.ds(dg * 16, 16)] for dg in range(DG)]

      @plsc.parallel_loop(0, SBB, unroll=8)
      def jj_body(jj):
        cl = jnp.full((16,), jj >> 7, jnp.int32)
        lane = jnp.full((16,), jj & 127, jnp.int32)
        for dg in range(DG):
          val = gbuf[jj, pl.ds(dg * 16, 16)] + posvecs[dg]
          plsc.store_scatter(tbuf, [gvecs[dg], cl, rvecs[dg], lane], val)

      pltpu.async_copy(tbuf.at[:, :, :, pl.ds(0, 128)],
                       out_hbm.at[l, :, pl.ds(c_start, CB), :, :], wsem)

    fire(bufs[0], 0)
    fire(bufs[1], 1)

    process(bufs[0], 0, True)
    fire(bufs[0], 2)
    process(bufs[1], 1, False)
    fire(bufs[1], 3)

    def pair_body(i, carry):
      k = 2 * i
      process(bufs[0], k, False)
      fire(bufs[0], k + 2)
      process(bufs[1], k + 1, False)
      fire(bufs[1], k + 3)
      return carry

    lax.fori_loop(1, SB_PER_W // 2 - 1, pair_body, 0)

    last = SB_PER_W - 2
    process(bufs[0], last, False)
    process(bufs[1], last + 1, False)
    wait_write()

  return emb_kernel


_emb = _build()


def kernel(inputs, token_table, pos_table):
  idx2d = inputs.astype(jnp.int32).T.reshape(-1, GIDX)
  W = _emb(idx2d, token_table, pos_table)
  return W.transpose(2, 4, 0, 1, 3).reshape(BATCH, MAX_LEN, EMBED_DIM)


# speedup vs baseline: 1.6719x; 1.0029x over previous
import functools

import jax
import jax.numpy as jnp
from jax import lax
from jax.experimental import pallas as pl
from jax.experimental.pallas import tpu as pltpu
from jax.experimental.pallas import tpu_sc as plsc

VOCAB = 1000000
MAX_LEN = 200
EMBED_DIM = 64
BATCH = 4096

NUM_CORES = 2
NUM_SUBCORES = 16
NW = NUM_CORES * NUM_SUBCORES
CB = 4
SBB = CB * 128
NSB = MAX_LEN * (BATCH // SBB)
SB_PER_W = NSB // NW
GIDX = 256
NG = SBB // GIDX
DG = EMBED_DIM // 16
C4 = BATCH // SBB


def _build():
  mesh = plsc.VectorSubcoreMesh(core_axis_name="c", subcore_axis_name="s")

  @functools.partial(
      pl.kernel,
      mesh=mesh,
      compiler_params=pltpu.CompilerParams(use_tc_tiling_on_sc=False,
                                           needs_layout_passes=False),
      out_type=jax.ShapeDtypeStruct((MAX_LEN, 8, BATCH // 128, 8, 128),
                                    jnp.float32),
      scratch_types=[
          pltpu.VMEM((NG, GIDX), jnp.int32),
          pltpu.VMEM((NG, GIDX), jnp.int32),
          pltpu.VMEM((SBB, EMBED_DIM), jnp.float32),
          pltpu.VMEM((SBB, EMBED_DIM), jnp.float32),
          pltpu.VMEM((8, CB, 8, 144), jnp.float32),
          pltpu.VMEM((MAX_LEN, EMBED_DIM), jnp.float32),
          pltpu.SemaphoreType.DMA,
          pltpu.SemaphoreType.DMA,
          pltpu.SemaphoreType.DMA,
      ],
  )
  def emb_kernel(idx_hbm, table_hbm, pos_hbm, out_hbm,
                 idx_a, idx_b, g_a, g_b, tbuf, pos_v,
                 gsem_a, gsem_b, wsem):
    wid = lax.axis_index("s") * NUM_CORES + lax.axis_index("c")
    sb0 = wid * SB_PER_W

    pltpu.sync_copy(pos_hbm, pos_v)

    iota16 = lax.iota(jnp.int32, 16)
    gvecs = [(dg * 16 + iota16) >> 3 for dg in range(DG)]
    rvecs = [(dg * 16 + iota16) & 7 for dg in range(DG)]

    bufs = ((idx_a, g_a, gsem_a), (idx_b, g_b, gsem_b))

    def fire(buf, i):
      idx_v, gbuf, gsem = buf
      pltpu.sync_copy(idx_hbm.at[pl.ds((sb0 + i) * NG, NG)], idx_v)
      for js in range(NG):
        pltpu.async_copy(
            table_hbm.at[idx_v.at[js]],
            gbuf.at[pl.ds(js * GIDX, GIDX)], gsem)

    def wait_write():
      pltpu.make_async_copy(tbuf.at[:, :, :, pl.ds(0, 128)],
                            out_hbm.at[0, :, pl.ds(0, CB), :, :],
                            wsem).wait()

    def process(buf, i, first):
      idx_v, gbuf, gsem = buf
      sbid = sb0 + i
      l = sbid // C4
      c_start = (sbid - l * C4) * CB
      for js in range(NG):
        pltpu.make_async_copy(
            table_hbm.at[idx_v.at[js]],
            gbuf.at[pl.ds(js * GIDX, GIDX)], gsem).wait()
      if not first:
        wait_write()
      posvecs = [pos_v[l, pl.ds(dg * 16, 16)] for dg in range(DG)]

      @plsc.parallel_loop(0, SBB, unroll=8)
      def jj_body(jj):
        cl = jnp.full((16,), jj >> 7, jnp.int32)
        lane = jnp.full((16,), jj & 127, jnp.int32)
        for dg in range(DG):
          val = gbuf[jj, pl.ds(dg * 16, 16)] + posvecs[dg]
          plsc.store_scatter(tbuf, [gvecs[dg], cl, rvecs[dg], lane], val)

      pltpu.async_copy(tbuf.at[:, :, :, pl.ds(0, 128)],
                       out_hbm.at[l, :, pl.ds(c_start, CB), :, :], wsem)

    fire(bufs[0], 0)
    fire(bufs[1], 1)

    process(bufs[0], 0, True)
    fire(bufs[0], 2)
    process(bufs[1], 1, False)
    fire(bufs[1], 3)

    def pair_body(i, carry):
      k = 2 * i
      process(bufs[0], k, False)
      fire(bufs[0], k + 2)
      process(bufs[1], k + 1, False)
      fire(bufs[1], k + 3)
      return carry

    lax.fori_loop(1, SB_PER_W // 2 - 1, pair_body, 0)

    last = SB_PER_W - 2
    process(bufs[0], last, False)
    process(bufs[1], last + 1, False)
    wait_write()

  return emb_kernel


_emb = _build()


def kernel(inputs, token_table, pos_table):
  idx2d = inputs.astype(jnp.int32).T.reshape(-1, GIDX)
  W = _emb(idx2d, token_table, pos_table)
  return W.transpose(2, 4, 0, 1, 3).reshape(BATCH, MAX_LEN, EMBED_DIM)
